# Initial kernel scaffold; baseline (speedup 1.0000x reference)
import jax
import jax.numpy as jnp
from jax.experimental import pallas as pl


def kernel(pred_logits, gts, step):
    raise NotImplementedError("write your pallas kernel here")


# speedup vs baseline: 34.8376x; 34.8376x over previous
import functools

import jax
import jax.numpy as jnp
from jax import lax
from jax.experimental import pallas as pl
from jax.experimental.pallas import tpu as pltpu
from jax.experimental.pallas import tpu_sc as plsc

_HW = 512 * 512
_TOTAL = 16 * 3 * _HW
_NW = 32
_PER_W = _TOTAL // _NW
_CHUNK = 16384
_NCH = _PER_W // _CHUNK
_N = float(_TOTAL)


def _bce_weighted(xv, yv, col_f):
    e = jnp.exp(-jnp.abs(xv))
    z = e / (2.0 + e)
    z2 = z * z
    l1p = 2.0 * z * (1.0 + z2 * (
        (1.0 / 3.0) + z2 * (0.2 + z2 * ((1.0 / 7.0) + z2 * (1.0 / 9.0)))))
    loss = jnp.maximum(xv, 0.0) - xv * yv + l1p
    return col_f * loss


def _make_sc_call():
    mesh = plsc.VectorSubcoreMesh(core_axis_name="c", subcore_axis_name="s")

    @functools.partial(
        pl.kernel,
        mesh=mesh,
        out_type=jax.ShapeDtypeStruct((_NW, 16), jnp.float32),
        scratch_types=[
            pltpu.VMEM((_CHUNK,), jnp.float32),
            pltpu.VMEM((_CHUNK,), jnp.float32),
            pltpu.VMEM((16,), jnp.float32),
        ],
    )
    def sc_call(x_hbm, y_hbm, out_hbm, xbuf, ybuf, accbuf):
        cid = lax.axis_index("c")
        sid = lax.axis_index("s")
        wid = sid * 2 + cid
        base = wid * _PER_W
        lane = lax.iota(jnp.int32, 16)

        def chunk_body(ci, acc):
            off = base + ci * _CHUNK
            pltpu.sync_copy(x_hbm.at[pl.ds(off, _CHUNK)], xbuf)
            pltpu.sync_copy(y_hbm.at[pl.ds(off, _CHUNK)], ybuf)

            def slice_body(i, acc_in):
                xv = xbuf[pl.ds(i * 16, 16)]
                yv = ybuf[pl.ds(i * 16, 16)]
                g = (off + i * 16) + lane
                col_f = jnp.bitwise_and(g, _HW - 1).astype(jnp.float32)
                return acc_in + _bce_weighted(xv, yv, col_f)

            return lax.fori_loop(0, _CHUNK // 16, slice_body, acc)

        acc = lax.fori_loop(0, _NCH, chunk_body, jnp.zeros((16,), jnp.float32))
        accbuf[...] = acc
        pltpu.sync_copy(accbuf, out_hbm.at[wid])

    return sc_call


_sc_call = _make_sc_call()


def kernel(pred_logits, gts, step):
    del step
    x = pred_logits.reshape(_TOTAL)
    y = gts.reshape(_TOTAL)
    partials = _sc_call(x, y)
    return jnp.sum(partials) * (1.0 / _N)


# speedup vs baseline: 45.6622x; 1.3107x over previous
import functools

import jax
import jax.numpy as jnp
from jax import lax
from jax.experimental import pallas as pl
from jax.experimental.pallas import tpu as pltpu
from jax.experimental.pallas import tpu_sc as plsc

_HW = 512 * 512
_TOTAL = 16 * 3 * _HW
_NW = 32
_PER_W = _TOTAL // _NW
_CHUNK = 16384
_NCH = _PER_W // _CHUNK
_UNROLL = 4
_N = float(_TOTAL)


def _bce_weighted(xv, yv, col_f):
    e = jnp.exp(-jnp.abs(xv))
    z = e / (2.0 + e)
    z2 = z * z
    l1p = 2.0 * z * (1.0 + z2 * (
        (1.0 / 3.0) + z2 * (0.2 + z2 * ((1.0 / 7.0) + z2 * (1.0 / 9.0)))))
    loss = jnp.maximum(xv, 0.0) - xv * yv + l1p
    return col_f * loss


def _make_sc_call():
    mesh = plsc.VectorSubcoreMesh(core_axis_name="c", subcore_axis_name="s")

    @functools.partial(
        pl.kernel,
        mesh=mesh,
        out_type=jax.ShapeDtypeStruct((_NW, 16), jnp.float32),
        scratch_types=[
            pltpu.VMEM((_CHUNK,), jnp.float32),
            pltpu.VMEM((_CHUNK,), jnp.float32),
            pltpu.VMEM((_CHUNK,), jnp.float32),
            pltpu.VMEM((_CHUNK,), jnp.float32),
            pltpu.VMEM((16,), jnp.float32),
            pltpu.SemaphoreType.DMA,
            pltpu.SemaphoreType.DMA,
        ],
    )
    def sc_call(x_hbm, y_hbm, out_hbm, xb0, yb0, xb1, yb1, accbuf, sem0, sem1):
        cid = lax.axis_index("c")
        sid = lax.axis_index("s")
        wid = sid * 2 + cid
        base = wid * _PER_W
        lane = lax.iota(jnp.int32, 16)

        def _start(ci, xb, yb, sem):
            off = base + ci * _CHUNK
            pltpu.make_async_copy(x_hbm.at[pl.ds(off, _CHUNK)], xb, sem).start()
            pltpu.make_async_copy(y_hbm.at[pl.ds(off, _CHUNK)], yb, sem).start()

        def _wait(xb, yb, sem):
            pltpu.make_async_copy(x_hbm.at[pl.ds(0, _CHUNK)], xb, sem).wait()
            pltpu.make_async_copy(y_hbm.at[pl.ds(0, _CHUNK)], yb, sem).wait()

        def _compute(ci, xb, yb, accs):
            off = base + ci * _CHUNK

            def inner(i, accs_in):
                outs = []
                for u in range(_UNROLL):
                    idx = i * (16 * _UNROLL) + u * 16
                    xv = xb[pl.ds(idx, 16)]
                    yv = yb[pl.ds(idx, 16)]
                    g = (off + idx) + lane
                    col_f = jnp.bitwise_and(g, _HW - 1).astype(jnp.float32)
                    outs.append(accs_in[u] + _bce_weighted(xv, yv, col_f))
                return tuple(outs)

            return lax.fori_loop(0, _CHUNK // (16 * _UNROLL), inner, accs)

        _start(0, xb0, yb0, sem0)
        zero = jnp.zeros((16,), jnp.float32)

        def outer(k, accs):
            c0 = 2 * k
            _start(c0 + 1, xb1, yb1, sem1)
            _wait(xb0, yb0, sem0)
            accs = _compute(c0, xb0, yb0, accs)

            @pl.when(c0 + 2 < _NCH)
            def _():
                _start(c0 + 2, xb0, yb0, sem0)

            _wait(xb1, yb1, sem1)
            return _compute(c0 + 1, xb1, yb1, accs)

        accs = lax.fori_loop(0, _NCH // 2, outer, (zero,) * _UNROLL)
        accbuf[...] = (accs[0] + accs[1]) + (accs[2] + accs[3])
        pltpu.sync_copy(accbuf, out_hbm.at[wid])

    return sc_call


_sc_call = _make_sc_call()


def kernel(pred_logits, gts, step):
    del step
    x = pred_logits.reshape(_TOTAL)
    y = gts.reshape(_TOTAL)
    partials = _sc_call(x, y)
    return jnp.sum(partials) * (1.0 / _N)
